# baseline (speedup 1.0000x reference)
import jax
import jax.numpy as jnp
from jax.experimental import pallas as pl


def _bn(h, gamma, beta, eps=1e-5):
    mu = jnp.mean(h, axis=0, keepdims=True)
    var = jnp.var(h, axis=0, keepdims=True)
    return gamma * (h - mu) / jnp.sqrt(var + eps) + beta


def _gen_conv(x, src, dst, p, eps=1e-7):
    n = x.shape[0]
    msg = jax.nn.relu(x[src]) + eps
    w = msg * p['t']
    m = jax.ops.segment_max(w, dst, num_segments=n)
    m = jnp.where(jnp.isneginf(m), 0.0, m)
    ew = jnp.exp(w - m[dst])
    denom = jax.ops.segment_sum(ew, dst, num_segments=n)
    alpha = ew / (denom[dst] + 1e-16)
    agg = jax.ops.segment_sum(msg * alpha, dst, num_segments=n)
    out = agg + x
    h = out @ p['W1'] + p['b1']
    h = _bn(h, p['gamma'], p['beta'])
    h = jax.nn.relu(h)
    return h @ p['W2'] + p['b2']


def _sage_conv(x, src, dst, p):
    n = x.shape[0]
    s = jax.ops.segment_sum(x[src], dst, num_segments=n)
    cnt = jax.ops.segment_sum(jnp.ones((src.shape[0], 1), x.dtype), dst, num_segments=n)
    mean = s / jnp.maximum(cnt, 1.0)
    return mean @ p['Wl'] + p['bl'] + x @ p['Wr']


def _identity_pallas(a):
    def body(a_ref, o_ref):
        o_ref[...] = a_ref[...]
    return pl.pallas_call(
        body, out_shape=jax.ShapeDtypeStruct(a.shape, a.dtype))(a)


def kernel(x, edge_index, params):
    src = edge_index[0]
    dst = edge_index[1]
    x0 = x
    h = x
    for p in params['convs']:
        h = _gen_conv(h, src, dst, p)
        h = jnp.concatenate([h, x0], axis=1)
    out = _sage_conv(h, src, dst, params['sage'])
    return _identity_pallas(out)


# speedup vs baseline: 24.5451x; 24.5451x over previous
import functools

import jax
import jax.numpy as jnp
from jax import lax
from jax.experimental import pallas as pl
from jax.experimental.pallas import tpu as pltpu
from jax.experimental.pallas import tpu_sc as plsc

N = 50000
E = 800000
N_PAD = 51200
RB = 2048
GRID = N_PAD // RB
NW = 32
WPT = 200
E_PAD = NW * WPT * 128
RING = 8
EPS = 1e-7


def _chunks(w):
    out = []
    while w > 0:
        c = min(32, w)
        out.append(c)
        w -= c
    return out


def _colmax_kernel(h_ref, t_ref, o_ref):
    i = pl.program_id(0)
    row = i * RB + lax.broadcasted_iota(jnp.int32, (RB, 1), 0)
    msg = jnp.maximum(h_ref[...], 0.0) + EPS
    w = msg * t_ref[0, 0]
    w = jnp.where(row < N, w, -3.4e38)
    bm = jnp.max(w, axis=0, keepdims=True)

    @pl.when(i == 0)
    def _():
        o_ref[...] = bm

    @pl.when(i > 0)
    def _():
        o_ref[...] = jnp.maximum(o_ref[...], bm)


def _tc_colmax(h, t):
    cin = h.shape[1]
    return pl.pallas_call(
        _colmax_kernel,
        grid=(GRID,),
        in_specs=[pl.BlockSpec((RB, cin), lambda i: (i, 0)),
                  pl.BlockSpec((1, 1), lambda i: (0, 0))],
        out_specs=pl.BlockSpec((1, cin), lambda i: (0, 0)),
        out_shape=jax.ShapeDtypeStruct((1, cin), jnp.float32),
    )(h, t.reshape(1, 1))


def _prep_kernel(widths, h_ref, m_ref, t_ref, *o_refs):
    i = pl.program_id(0)
    row = i * RB + lax.broadcasted_iota(jnp.int32, (RB, 1), 0)
    msg = jnp.maximum(h_ref[...], 0.0) + EPS
    w = msg * t_ref[0, 0]
    ew = jnp.exp(w - m_ref[...])
    p = jnp.concatenate([ew, msg * ew], axis=1)
    p = jnp.where(row < N, p, 0.0)
    c0 = 0
    for j, wd in enumerate(widths):
        o_refs[j][...] = p[:, c0:c0 + wd]
        c0 += wd


def _tc_prep(h, m, t, widths):
    cin = h.shape[1]
    return pl.pallas_call(
        functools.partial(_prep_kernel, widths),
        grid=(GRID,),
        in_specs=[pl.BlockSpec((RB, cin), lambda i: (i, 0)),
                  pl.BlockSpec((1, cin), lambda i: (0, 0)),
                  pl.BlockSpec((1, 1), lambda i: (0, 0))],
        out_specs=[pl.BlockSpec((RB, wd), lambda i: (i, 0)) for wd in widths],
        out_shape=[jax.ShapeDtypeStruct((N_PAD, wd), jnp.float32)
                   for wd in widths],
    )(h, m, t.reshape(1, 1))


def _posta_kernel(nchunk, cin, h_ref, w1_ref, b1_ref, *a_refs):
    agg_refs = a_refs[:nchunk]
    h1_ref, st_ref = a_refs[nchunk], a_refs[nchunk + 1]
    i = pl.program_id(0)
    row = i * RB + lax.broadcasted_iota(jnp.int32, (RB, 1), 0)
    parts = [r[0] + r[1] for r in agg_refs]
    full = jnp.concatenate(parts, axis=1)
    den = full[:, :cin]
    num = full[:, cin:2 * cin]
    out = num / (den + 1e-16) + h_ref[...]
    h1 = jnp.dot(out, w1_ref[...], preferred_element_type=jnp.float32) \
        + b1_ref[...]
    h1_ref[...] = h1
    hm = jnp.where(row < N, h1, 0.0)
    s1 = jnp.sum(hm, axis=0, keepdims=True)
    s2 = jnp.sum(hm * hm, axis=0, keepdims=True)
    blk = jnp.concatenate([s1, s2], axis=0)

    @pl.when(i == 0)
    def _():
        st_ref[...] = blk

    @pl.when(i > 0)
    def _():
        st_ref[...] = st_ref[...] + blk


def _tc_posta(aggs, h, w1, b1):
    cin = h.shape[1]
    hid = w1.shape[1]
    widths = [a.shape[2] for a in aggs]
    return pl.pallas_call(
        functools.partial(_posta_kernel, len(aggs), cin),
        grid=(GRID,),
        in_specs=[pl.BlockSpec((RB, cin), lambda i: (i, 0)),
                  pl.BlockSpec((cin, hid), lambda i: (0, 0)),
                  pl.BlockSpec((1, hid), lambda i: (0, 0))]
        + [pl.BlockSpec((2, RB, wd), lambda i: (0, i, 0)) for wd in widths],
        out_specs=[pl.BlockSpec((RB, hid), lambda i: (i, 0)),
                   pl.BlockSpec((2, hid), lambda i: (0, 0))],
        out_shape=[jax.ShapeDtypeStruct((N_PAD, hid), jnp.float32),
                   jax.ShapeDtypeStruct((2, hid), jnp.float32)],
    )(h, w1, b1.reshape(1, hid), *aggs)


def _postb_kernel(h1_ref, st_ref, g_ref, be_ref, w2_ref, b2_ref, x0_ref,
                  o_ref):
    mu = st_ref[0:1, :] * (1.0 / N)
    var = st_ref[1:2, :] * (1.0 / N) - mu * mu
    h = (h1_ref[...] - mu) / jnp.sqrt(var + 1e-5) * g_ref[...] + be_ref[...]
    h = jnp.maximum(h, 0.0)
    h2 = jnp.dot(h, w2_ref[...], preferred_element_type=jnp.float32) \
        + b2_ref[...]
    o_ref[...] = jnp.concatenate([h2, x0_ref[...]], axis=1)


def _tc_postb(h1, st, gamma, beta, w2, b2, x0):
    hid = h1.shape[1]
    cout = w2.shape[1]
    c2 = cout + x0.shape[1]
    return pl.pallas_call(
        _postb_kernel,
        grid=(GRID,),
        in_specs=[pl.BlockSpec((RB, hid), lambda i: (i, 0)),
                  pl.BlockSpec((2, hid), lambda i: (0, 0)),
                  pl.BlockSpec((1, hid), lambda i: (0, 0)),
                  pl.BlockSpec((1, hid), lambda i: (0, 0)),
                  pl.BlockSpec((hid, cout), lambda i: (0, 0)),
                  pl.BlockSpec((1, cout), lambda i: (0, 0)),
                  pl.BlockSpec((RB, x0.shape[1]), lambda i: (i, 0))],
        out_specs=pl.BlockSpec((RB, c2), lambda i: (i, 0)),
        out_shape=jax.ShapeDtypeStruct((N_PAD, c2), jnp.float32),
    )(h1, st, gamma.reshape(1, hid), beta.reshape(1, hid), w2,
      b2.reshape(1, cout), x0)


def _sageprep_kernel(h_ref, o_ref):
    i = pl.program_id(0)
    row = i * RB + lax.broadcasted_iota(jnp.int32, (RB, 1), 0)
    ones = jnp.ones((RB, 1), jnp.float32)
    zer = jnp.zeros((RB, 3), jnp.float32)
    p = jnp.concatenate([h_ref[...], ones, zer], axis=1)
    o_ref[...] = jnp.where(row < N, p, 0.0)


def _tc_sageprep(h):
    cin = h.shape[1]
    return pl.pallas_call(
        _sageprep_kernel,
        grid=(GRID,),
        in_specs=[pl.BlockSpec((RB, cin), lambda i: (i, 0))],
        out_specs=pl.BlockSpec((RB, 16), lambda i: (i, 0)),
        out_shape=jax.ShapeDtypeStruct((N_PAD, 16), jnp.float32),
    )(h)


def _sage_kernel(a_ref, h_ref, wl_ref, bl_ref, wr_ref, o_ref):
    s = a_ref[0] + a_ref[1]
    ssum = s[:, :12]
    cnt = s[:, 12:13]
    mean = ssum / jnp.maximum(cnt, 1.0)
    o_ref[...] = (jnp.dot(mean, wl_ref[...], preferred_element_type=jnp.float32)
                  + bl_ref[...]
                  + jnp.dot(h_ref[...], wr_ref[...],
                            preferred_element_type=jnp.float32))


def _tc_sage(agg, h, wl, bl, wr):
    return pl.pallas_call(
        _sage_kernel,
        grid=(GRID,),
        in_specs=[pl.BlockSpec((2, RB, 16), lambda i: (0, i, 0)),
                  pl.BlockSpec((RB, 12), lambda i: (i, 0)),
                  pl.BlockSpec((12, 1), lambda i: (0, 0)),
                  pl.BlockSpec((1, 1), lambda i: (0, 0)),
                  pl.BlockSpec((12, 1), lambda i: (0, 0))],
        out_specs=pl.BlockSpec((RB, 1), lambda i: (i, 0)),
        out_shape=jax.ShapeDtypeStruct((N_PAD, 1), jnp.float32),
    )(agg, h, wl, bl.reshape(1, 1), wr)


NI = 8
NG = 4


def _sc_body(wc, p_hbm, ed_hbm, z_hbm, out_hbm, acc, *rest):
    eds = rest[:NI]
    rows = rest[NI:NI + NG]
    semi = rest[NI + NG:NI + NG + NI]
    semg = rest[NI + NG + NI:]
    cid = lax.axis_index("c")
    sid = lax.axis_index("s")
    wid = cid * 16 + sid
    base = sid * 3200

    def start_idx(g, j):
        pltpu.make_async_copy(ed_hbm.at[wid, g], eds[j], semi[j]).start()

    def wait_idx(j):
        pltpu.make_async_copy(ed_hbm.at[wid, 0], eds[j], semi[j]).wait()

    def start_gather(j4, j8):
        pltpu.make_async_copy(p_hbm.at[eds[j8].at[0]], rows[j4],
                              semg[j4]).start()

    def wait_gather(j4):
        pltpu.make_async_copy(z_hbm, rows[j4], semg[j4]).wait()

    pltpu.sync_copy(z_hbm, rows[0])

    def zbody(r, _):
        pltpu.sync_copy(rows[0], acc.at[pl.ds(base + r * 128, 128)])
        return ()

    lax.fori_loop(0, N_PAD // (16 * 128), zbody, ())
    plsc.subcore_barrier()

    for j in range(NG):
        start_idx(j, j)

    def mbody(blk, _):
        for v in range(8):
            g = blk * 8 + v

            @pl.when(jnp.logical_and(g >= NG, g < WPT + NG))
            def _():
                wait_gather(v % NG)
                pltpu.sync_copy(rows[v % NG],
                                acc.at[eds[(v - NG) % NI].at[1]], add=True)

            @pl.when(g < WPT)
            def _():
                wait_idx(v % NI)
                start_gather(v % NG, v % NI)

            @pl.when(g + NG < WPT)
            def _():
                start_idx(g + NG, (v + NG) % NI)
        return ()

    lax.fori_loop(0, (WPT + NG + 7) // 8, mbody, ())
    plsc.subcore_barrier()

    def dbody(r, _):
        sl = pl.ds(base + r * 128, 128)
        pltpu.sync_copy(acc.at[sl], rows[0])
        pltpu.sync_copy(rows[0], out_hbm.at[cid, sl])
        return ()

    lax.fori_loop(0, N_PAD // (16 * 128), dbody, ())


@functools.cache
def _sc_pass(wc):
    mesh = plsc.VectorSubcoreMesh(core_axis_name="c", subcore_axis_name="s")
    return pl.kernel(
        functools.partial(_sc_body, wc),
        mesh=mesh,
        compiler_params=pltpu.CompilerParams(use_tc_tiling_on_sc=False),
        out_type=jax.ShapeDtypeStruct((2, N_PAD, wc), jnp.float32),
        scratch_types=[
            pltpu.VMEM_SHARED((N_PAD, wc), jnp.float32),
        ] + [pltpu.VMEM((2, 128), jnp.int32) for _ in range(NI)]
        + [pltpu.VMEM((128, wc), jnp.float32) for _ in range(NG)]
        + [pltpu.SemaphoreType.DMA for _ in range(NI + NG)],
    )


def _sc_aggregate(p_chunks, edw):
    outs = []
    for p in p_chunks:
        wc = p.shape[1]
        z = jnp.zeros((128, wc), jnp.float32)
        outs.append(_sc_pass(wc)(p, edw, z))
    return outs


def kernel(x, edge_index, params):
    src = edge_index[0].astype(jnp.int32)
    dst = edge_index[1].astype(jnp.int32)
    npad = E_PAD - E
    pad_idx = N + (jnp.arange(npad, dtype=jnp.int32) % (N_PAD - N))
    srcw = jnp.concatenate([src, pad_idx]).reshape(NW, WPT, 1, 128)
    dstw = jnp.concatenate([dst, pad_idx]).reshape(NW, WPT, 1, 128)
    edw = jnp.concatenate([srcw, dstw], axis=2)

    xp = jnp.zeros((N_PAD, 4), jnp.float32).at[:N].set(x)
    h = xp
    for p in params['convs']:
        cin = h.shape[1]
        widths = _chunks(2 * cin)
        m = _tc_colmax(h, p['t'])
        pch = _tc_prep(h, m, p['t'], widths)
        aggs = _sc_aggregate(pch, edw)
        h1, st = _tc_posta(aggs, h, p['W1'], p['b1'])
        h = _tc_postb(h1, st, p['gamma'], p['beta'], p['W2'], p['b2'], xp)

    ps = _tc_sageprep(h)
    agg = _sc_aggregate([ps], edw)[0]
    out = _tc_sage(agg, h, params['sage']['Wl'], params['sage']['bl'],
                   params['sage']['Wr'])
    return out[:N]
